# baseline (speedup 1.0000x reference)
import functools

import jax
import jax.numpy as jnp
from jax import lax
from jax.experimental import pallas as pl
from jax.experimental.pallas import tpu as pltpu
from jax.experimental.pallas import tpu_sc as plsc

NUM_CORES = 2
NUM_SUBCORES = 16
NUM_WORKERS = NUM_CORES * NUM_SUBCORES
LANES = 16

CHUNK = 128
NBUF = 4


def _sc_partial_sums(ids_flat, table, B, L, D):
  ids_per_worker = (B * L) // NUM_WORKERS
  num_chunks = ids_per_worker // CHUNK
  vecs_per_row = D // LANES
  log2_b = B.bit_length() - 1
  assert (1 << log2_b) == B

  mesh = plsc.VectorSubcoreMesh(
      core_axis_name="c", subcore_axis_name="s",
      num_cores=NUM_CORES, num_subcores=NUM_SUBCORES)

  scratch = (
      [pltpu.VMEM((ids_per_worker,), jnp.int32)]
      + [pltpu.VMEM((CHUNK, D), jnp.float32) for _ in range(NBUF)]
      + [pltpu.VMEM((L, D), jnp.float32)]
      + [pltpu.SemaphoreType.DMA for _ in range(NBUF)]
  )

  @functools.partial(
      pl.kernel,
      out_type=jax.ShapeDtypeStruct((NUM_WORKERS, L, D), jnp.float32),
      mesh=mesh,
      scratch_types=scratch,
      compiler_params=pltpu.CompilerParams(use_tc_tiling_on_sc=False),
  )
  def body(ids_hbm, table_hbm, out_hbm, *refs):
    idx_v = refs[0]
    rows = refs[1:1 + NBUF]
    acc_v = refs[1 + NBUF]
    sems = refs[2 + NBUF:2 + 2 * NBUF]

    wid = lax.axis_index("s") * NUM_CORES + lax.axis_index("c")
    start = wid * ids_per_worker

    pltpu.sync_copy(ids_hbm.at[pl.ds(start, ids_per_worker)], idx_v)

    zero = jnp.zeros((LANES,), jnp.float32)
    for r in range(L):
      for h in range(vecs_per_row):
        acc_v[r, pl.ds(h * LANES, LANES)] = zero

    def chunk_idx(c):
      return idx_v.at[pl.ds(c * CHUNK, CHUNK)]

    for b in range(NBUF):
      pltpu.async_copy(table_hbm.at[chunk_idx(b)], rows[b], sems[b])

    def loop_body(it, carry):
      j = it * NBUF
      for b in range(NBUF):
        cur = j + b
        pltpu.make_async_copy(
            table_hbm.at[chunk_idx(cur)], rows[b], sems[b]).wait()
        l_dyn = (start + cur * CHUNK) >> log2_b
        accs = [zero] * (2 * vecs_per_row)
        for r in range(CHUNK):
          for h in range(vecs_per_row):
            a = (r % 2) * vecs_per_row + h
            accs[a] = accs[a] + rows[b][r, pl.ds(h * LANES, LANES)]
        for h in range(vecs_per_row):
          plsc.addupdate(acc_v.at[l_dyn, pl.ds(h * LANES, LANES)],
                         accs[h] + accs[vecs_per_row + h])
        nxt = cur + NBUF

        @pl.when(nxt < num_chunks)
        def _():
          pltpu.async_copy(table_hbm.at[chunk_idx(nxt)], rows[b], sems[b])
      return carry

    lax.fori_loop(0, num_chunks // NBUF, loop_body, 0, unroll=False)

    pltpu.sync_copy(acc_v, out_hbm.at[wid])

  return body(ids_flat, table)


def _tc_combine(partials, L, D):
  def body(x_ref, o_ref):
    o_ref[...] = jnp.sum(x_ref[...], axis=0)

  return pl.pallas_call(
      body,
      out_shape=jax.ShapeDtypeStruct((L, D), jnp.float32),
  )(partials)


def kernel(ingredient_ids, table):
  B, L = ingredient_ids.shape
  V, D = table.shape
  ids = ingredient_ids.astype(jnp.int32)

  ids_per_worker = (B * L) // NUM_WORKERS
  assert (B * L) % (NUM_WORKERS * CHUNK) == 0
  assert B % CHUNK == 0 and D % LANES == 0

  ids_flat = ids.T.reshape(-1)
  partials = _sc_partial_sums(ids_flat, table, B, L, D)
  return _tc_combine(partials, L, D)
